# Initial kernel scaffold; baseline (speedup 1.0000x reference)
import jax
import jax.numpy as jnp
from jax.experimental import pallas as pl


def kernel(x, trainX, trainy, ymean, l, a):
    raise NotImplementedError("write your pallas kernel here")


# speedup vs baseline: 1.4991x; 1.4991x over previous
import jax
import jax.numpy as jnp
from jax.experimental import pallas as pl
from jax.experimental.pallas import tpu as pltpu

_NN = 128
_QB = 128
_TB = 512
_CG_ITERS = 64
_GPB = 8


def _dist_kernel(xs_ref, ts_ref, xn_ref, tn_ref, out_ref):
    mm = jax.lax.dot_general(
        xs_ref[...], ts_ref[...], (((1,), (1,)), ((), ())),
        preferred_element_type=jnp.float32)
    d2 = (xn_ref[...] + tn_ref[...]) - 2.0 * mm
    out_ref[...] = jnp.sqrt(jnp.maximum(d2, 1e-12))


def _gp_kernel(xq_ref, nx_ref, nyc_ref, out_ref):
    b_n = xq_ref.shape[0]
    nx = nx_ref[...]
    xq = xq_ref[...]
    nyc = nyc_ref[...]

    grams = []
    crosses = []
    for b in range(b_n):
        nxb = nx[b]
        grams.append(jax.lax.dot_general(
            nxb, nxb, (((1,), (1,)), ((), ())),
            preferred_element_type=jnp.float32))
        crosses.append(jax.lax.dot_general(
            nxb, xq[b:b + 1], (((1,), (1,)), ((), ())),
            preferred_element_type=jnp.float32)[:, 0])
    gram = jnp.stack(grams)
    nxq = jnp.stack(crosses)

    rn = jnp.sum(nx * nx, axis=2)
    qn = jnp.sum(xq * xq, axis=1)
    d2 = rn[:, :, None] + rn[:, None, :] - 2.0 * gram
    eg = jnp.exp(-jnp.sqrt(jnp.maximum(d2, 1e-12)) * 0.125)
    qd2 = rn + qn[:, None] - 2.0 * nxq
    e = jnp.exp(-jnp.sqrt(jnp.maximum(qd2, 1e-12)) * 0.125)

    def matvec(v):
        return jnp.sum(eg * v[:, None, :], axis=2)

    def cg_pair_body(_, c):
        z1, r1, p1, rr1, z2, r2, p2, rr2 = c

        def step(z, r, p, rr):
            ap = matvec(p)
            pap = jnp.sum(p * ap, axis=1, keepdims=True)
            alpha = rr / jnp.maximum(pap, 1e-30)
            z = z + alpha * p
            r = r - alpha * ap
            rr2_ = jnp.sum(r * r, axis=1, keepdims=True)
            beta = rr2_ / jnp.maximum(rr, 1e-30)
            return z, r, r + beta * p, rr2_

        z1, r1, p1, rr1 = step(z1, r1, p1, rr1)
        z2, r2, p2, rr2 = step(z2, r2, p2, rr2)
        return z1, r1, p1, rr1, z2, r2, p2, rr2

    rr1_0 = jnp.sum(e * e, axis=1, keepdims=True)
    rr2_0 = jnp.sum(nyc * nyc, axis=1, keepdims=True)
    init = (jnp.zeros_like(e), e, e, rr1_0,
            jnp.zeros_like(nyc), nyc, nyc, rr2_0)
    z1, _, _, _, z2, _, _, _ = jax.lax.fori_loop(
        0, _CG_ITERS, cg_pair_body, init)

    y_raw = jnp.sum(e * z2, axis=1)
    v_raw = 1.0 - jnp.sum(e * z1, axis=1)
    lane = jax.lax.broadcasted_iota(jnp.int32, (b_n, _NN), 1)
    out_ref[...] = jnp.where(lane == 0, y_raw[:, None],
                             jnp.where(lane == 1, v_raw[:, None], 0.0))


def kernel(x, trainX, trainy, ymean, l, a):
    q, d = x.shape
    n = trainX.shape[0]
    le = jnp.exp(l)
    ae = jnp.exp(a)
    xs = x / le
    ts = trainX / le

    npad = ((n + _TB - 1) // _TB) * _TB
    tpad = jnp.pad(ts, ((0, npad - n), (0, 0)), constant_values=1e6)

    xn = jnp.sum(xs * xs, axis=-1)[:, None]
    tn = jnp.sum(tpad * tpad, axis=-1)[None, :]

    dists = pl.pallas_call(
        _dist_kernel,
        grid=(q // _QB, npad // _TB),
        in_specs=[pl.BlockSpec((_QB, d), lambda i, j: (i, 0)),
                  pl.BlockSpec((_TB, d), lambda i, j: (j, 0)),
                  pl.BlockSpec((_QB, 1), lambda i, j: (i, 0)),
                  pl.BlockSpec((1, _TB), lambda i, j: (0, j))],
        out_specs=pl.BlockSpec((_QB, _TB), lambda i, j: (i, j)),
        out_shape=jax.ShapeDtypeStruct((q, npad), jnp.float32),
    )(xs, tpad, xn, tn)

    _, idx = jax.lax.top_k(-dists, _NN + 1)
    nbr = idx[:, 1:]

    nxg = ts[nbr]
    noise = jax.random.normal(jax.random.key(1), (q, _NN, 1),
                              dtype=jnp.float32)
    nyc = trainy[nbr][:, :, 0] + 0.01 * noise[:, :, 0] - ymean

    out = pl.pallas_call(
        _gp_kernel,
        grid=(q // _GPB,),
        in_specs=[pl.BlockSpec((_GPB, 64), lambda i: (i, 0)),
                  pl.BlockSpec((_GPB, _NN, 64), lambda i: (i, 0, 0)),
                  pl.BlockSpec((_GPB, _NN), lambda i: (i, 0))],
        out_specs=pl.BlockSpec((_GPB, _NN), lambda i: (i, 0)),
        out_shape=jax.ShapeDtypeStruct((q, _NN), jnp.float32),
    )(xs, nxg, nyc)

    y = out[:, 0:1] + ymean
    yvar = ae * out[:, 1]
    return (y, yvar)


# speedup vs baseline: 1.6217x; 1.0818x over previous
import jax
import jax.numpy as jnp
from jax.experimental import pallas as pl
from jax.experimental.pallas import tpu as pltpu

_NN = 128
_QB = 128
_TB = 512
_GPB = 8


def _dist_kernel(xs_ref, ts_ref, xn_ref, tn_ref, out_ref):
    mm = jax.lax.dot_general(
        xs_ref[...], ts_ref[...], (((1,), (1,)), ((), ())),
        preferred_element_type=jnp.float32)
    d2 = (xn_ref[...] + tn_ref[...]) - 2.0 * mm
    out_ref[...] = jnp.sqrt(jnp.maximum(d2, 1e-12))


def _gp_kernel(xq_ref, nx_ref, nyc_ref, ae_ref, out_ref):
    b_n = xq_ref.shape[0]
    nx = nx_ref[...]
    xq = xq_ref[...]
    nyc = nyc_ref[...]
    aev = ae_ref[...]

    grams = []
    for b in range(b_n):
        nxb = nx[b]
        grams.append(jax.lax.dot_general(
            nxb, nxb, (((1,), (1,)), ((), ())),
            preferred_element_type=jnp.float32))
    gram = jnp.stack(grams)
    nx16 = nx.astype(jnp.bfloat16).astype(jnp.float32)
    xq16 = xq.astype(jnp.bfloat16).astype(jnp.float32)
    nxq = jnp.sum(nx16 * xq16[:, None, :], axis=2)

    rn = jnp.sum(nx * nx, axis=2)
    qn = jnp.sum(xq * xq, axis=1)
    d2 = rn[:, :, None] + rn[:, None, :] - 2.0 * gram
    eg = jnp.exp(-jnp.sqrt(jnp.maximum(d2, 1e-12)) * 0.125)
    qd2 = rn + qn[:, None] - 2.0 * nxq
    e = aev * jnp.exp(-jnp.sqrt(jnp.maximum(qd2, 1e-12)) * 0.125)

    rows = jax.lax.broadcasted_iota(jnp.int32, (1, _NN, 1), 1)
    lanes2 = jax.lax.broadcasted_iota(jnp.int32, (1, 1, 2 * _NN), 2)
    eye = jnp.asarray(
        jax.lax.broadcasted_iota(jnp.int32, (1, _NN, _NN), 1)
        == jax.lax.broadcasted_iota(jnp.int32, (1, _NN, _NN), 2))
    m0 = jnp.concatenate(
        [eg, jnp.broadcast_to(jnp.where(eye, 1.0, 0.0), (b_n, _NN, _NN))],
        axis=2)

    def gj_body(k, m):
        rmask = rows == k
        lmask = lanes2 == k
        prow = jnp.sum(jnp.where(rmask, m, 0.0), axis=1, keepdims=True)
        piv = jnp.sum(jnp.where(lmask, prow, 0.0), axis=2, keepdims=True)
        col = jnp.sum(jnp.where(lmask[:, :, :_NN], m[:, :, :_NN], 0.0),
                      axis=2, keepdims=True)
        f = jnp.where(rmask, 0.0, col * (1.0 / piv))
        return m - f * prow

    m = jax.lax.fori_loop(0, _NN, gj_body, m0)
    diag = jnp.sum(jnp.where(eye, m[:, :, :_NN], 0.0), axis=2)
    inv_raw = m[:, :, _NN:] / diag[:, :, None]

    def dot_h(u, v):
        return jax.lax.dot_general(
            u, v, (((1,), (0,)), ((), ())),
            precision=jax.lax.Precision.HIGHEST,
            preferred_element_type=jnp.float32)

    refined = []
    for b in range(b_n):
        xb = inv_raw[b]
        t = dot_h(eg[b], xb)
        refined.append(2.0 * xb - dot_h(xb, t))
    inv = jnp.stack(refined) / aev[:, :, None]

    ys = []
    ts = []
    e16 = e.astype(jnp.bfloat16)
    inv16 = inv.astype(jnp.bfloat16)
    for b in range(b_n):
        kw = jax.lax.dot_general(
            e16[b:b + 1], inv16[b], (((1,), (0,)), ((), ())),
            preferred_element_type=jnp.float32)
        yb = jnp.sum(kw * nyc[b:b + 1], axis=1, keepdims=True)
        tb = jnp.sum(kw * e[b:b + 1], axis=1, keepdims=True)
        ys.append(yb)
        ts.append(tb)
    y_raw = jnp.concatenate(ys, axis=0)
    t_raw = jnp.concatenate(ts, axis=0)
    lane = jax.lax.broadcasted_iota(jnp.int32, (b_n, _NN), 1)
    out_ref[...] = jnp.where(lane == 0, y_raw,
                             jnp.where(lane == 1, t_raw, 0.0))


def kernel(x, trainX, trainy, ymean, l, a):
    q, d = x.shape
    n = trainX.shape[0]
    le = jnp.exp(l)
    ae = jnp.exp(a)
    xs = x / le
    ts = trainX / le

    npad = ((n + _TB - 1) // _TB) * _TB
    tpad = jnp.pad(ts, ((0, npad - n), (0, 0)), constant_values=1e6)

    xn = jnp.sum(xs * xs, axis=-1)[:, None]
    tn = jnp.sum(tpad * tpad, axis=-1)[None, :]

    dists = pl.pallas_call(
        _dist_kernel,
        grid=(q // _QB, npad // _TB),
        in_specs=[pl.BlockSpec((_QB, d), lambda i, j: (i, 0)),
                  pl.BlockSpec((_TB, d), lambda i, j: (j, 0)),
                  pl.BlockSpec((_QB, 1), lambda i, j: (i, 0)),
                  pl.BlockSpec((1, _TB), lambda i, j: (0, j))],
        out_specs=pl.BlockSpec((_QB, _TB), lambda i, j: (i, j)),
        out_shape=jax.ShapeDtypeStruct((q, npad), jnp.float32),
    )(xs, tpad, xn, tn)

    _, idx = jax.lax.top_k(-dists, _NN + 1)
    nbr = idx[:, 1:]

    nxg = ts[nbr]
    noise = jax.random.normal(jax.random.key(1), (q, _NN, 1),
                              dtype=jnp.float32)
    nyc = trainy[nbr][:, :, 0] + 0.01 * noise[:, :, 0] - ymean

    aev = jnp.broadcast_to(ae[None, None], (q, 1)).astype(jnp.float32)
    out = pl.pallas_call(
        _gp_kernel,
        grid=(q // _GPB,),
        in_specs=[pl.BlockSpec((_GPB, 64), lambda i: (i, 0)),
                  pl.BlockSpec((_GPB, _NN, 64), lambda i: (i, 0, 0)),
                  pl.BlockSpec((_GPB, _NN), lambda i: (i, 0)),
                  pl.BlockSpec((_GPB, 1), lambda i: (i, 0))],
        out_specs=pl.BlockSpec((_GPB, _NN), lambda i: (i, 0)),
        out_shape=jax.ShapeDtypeStruct((q, _NN), jnp.float32),
    )(xs, nxg, nyc, aev)

    y = out[:, 0:1] + ymean
    yvar = ae - out[:, 1]
    return (y, yvar)


# speedup vs baseline: 1.6747x; 1.0327x over previous
import jax
import jax.numpy as jnp
from jax.experimental import pallas as pl
from jax.experimental.pallas import tpu as pltpu

_NN = 128
_QB = 128
_TB = 512
_GPB = 8


def _dist_kernel(xs_ref, ts_ref, xn_ref, tn_ref, out_ref):
    mm = jax.lax.dot_general(
        xs_ref[...], ts_ref[...], (((1,), (1,)), ((), ())),
        preferred_element_type=jnp.float32)
    d2 = (xn_ref[...] + tn_ref[...]) - 2.0 * mm
    out_ref[...] = jnp.sqrt(jnp.maximum(d2, 1e-12))


def _gp_kernel(xq_ref, nx_ref, nyc_ref, ae_ref, out_ref):
    b_n = xq_ref.shape[0]
    nx = nx_ref[...]
    xq = xq_ref[...]
    nyc = nyc_ref[...]
    aev = ae_ref[...]

    grams = []
    for b in range(b_n):
        nxb = nx[b]
        grams.append(jax.lax.dot_general(
            nxb, nxb, (((1,), (1,)), ((), ())),
            preferred_element_type=jnp.float32))
    gram = jnp.stack(grams)
    nx16 = nx.astype(jnp.bfloat16).astype(jnp.float32)
    xq16 = xq.astype(jnp.bfloat16).astype(jnp.float32)
    nxq = jnp.sum(nx16 * xq16[:, None, :], axis=2)

    rn = jnp.sum(nx * nx, axis=2)
    qn = jnp.sum(xq * xq, axis=1)
    d2 = rn[:, :, None] + rn[:, None, :] - 2.0 * gram
    eg = jnp.exp(-jnp.sqrt(jnp.maximum(d2, 1e-12)) * 0.125)
    qd2 = rn + qn[:, None] - 2.0 * nxq
    e = aev * jnp.exp(-jnp.sqrt(jnp.maximum(qd2, 1e-12)) * 0.125)

    rows = jax.lax.broadcasted_iota(jnp.int32, (1, _NN, 1), 1)
    lanes = jax.lax.broadcasted_iota(jnp.int32, (1, 1, _NN), 2)

    def gj_body(k, m):
        rmask = rows == k
        lmask = lanes == k
        prow = jnp.sum(jnp.where(rmask, m, 0.0), axis=1, keepdims=True)
        piv = jnp.sum(jnp.where(lmask, prow, 0.0), axis=2, keepdims=True)
        col = jnp.sum(jnp.where(lmask, m, 0.0), axis=2, keepdims=True)
        rp = 1.0 / piv
        prow2 = jnp.where(lmask, rp, prow * rp)
        f = jnp.where(rmask, 0.0, col)
        upd = m - f * (prow2 + jnp.where(lmask, 1.0, 0.0))
        return jnp.where(rmask, prow2, upd)

    inv_raw = jax.lax.fori_loop(0, _NN, gj_body, eg)

    def dot_h(u, v):
        return jax.lax.dot_general(
            u, v, (((1,), (0,)), ((), ())),
            precision=jax.lax.Precision.HIGHEST,
            preferred_element_type=jnp.float32)

    refined = []
    for b in range(b_n):
        xb = inv_raw[b]
        t = dot_h(eg[b], xb)
        refined.append(2.0 * xb - dot_h(xb, t))
    inv = jnp.stack(refined) / aev[:, :, None]

    ys = []
    ts = []
    e16 = e.astype(jnp.bfloat16)
    inv16 = inv.astype(jnp.bfloat16)
    for b in range(b_n):
        kw = jax.lax.dot_general(
            e16[b:b + 1], inv16[b], (((1,), (0,)), ((), ())),
            preferred_element_type=jnp.float32)
        yb = jnp.sum(kw * nyc[b:b + 1], axis=1, keepdims=True)
        tb = jnp.sum(kw * e[b:b + 1], axis=1, keepdims=True)
        ys.append(yb)
        ts.append(tb)
    y_raw = jnp.concatenate(ys, axis=0)
    t_raw = jnp.concatenate(ts, axis=0)
    lane = jax.lax.broadcasted_iota(jnp.int32, (b_n, _NN), 1)
    out_ref[...] = jnp.where(lane == 0, y_raw,
                             jnp.where(lane == 1, t_raw, 0.0))


def kernel(x, trainX, trainy, ymean, l, a):
    q, d = x.shape
    n = trainX.shape[0]
    le = jnp.exp(l)
    ae = jnp.exp(a)
    xs = x / le
    ts = trainX / le

    npad = ((n + _TB - 1) // _TB) * _TB
    tpad = jnp.pad(ts, ((0, npad - n), (0, 0)), constant_values=1e6)

    xn = jnp.sum(xs * xs, axis=-1)[:, None]
    tn = jnp.sum(tpad * tpad, axis=-1)[None, :]

    dists = pl.pallas_call(
        _dist_kernel,
        grid=(q // _QB, npad // _TB),
        in_specs=[pl.BlockSpec((_QB, d), lambda i, j: (i, 0)),
                  pl.BlockSpec((_TB, d), lambda i, j: (j, 0)),
                  pl.BlockSpec((_QB, 1), lambda i, j: (i, 0)),
                  pl.BlockSpec((1, _TB), lambda i, j: (0, j))],
        out_specs=pl.BlockSpec((_QB, _TB), lambda i, j: (i, j)),
        out_shape=jax.ShapeDtypeStruct((q, npad), jnp.float32),
    )(xs, tpad, xn, tn)

    _, idx = jax.lax.top_k(-dists, _NN + 1)
    nbr = idx[:, 1:]

    nxg = ts[nbr]
    noise = jax.random.normal(jax.random.key(1), (q, _NN, 1),
                              dtype=jnp.float32)
    nyc = trainy[nbr][:, :, 0] + 0.01 * noise[:, :, 0] - ymean

    aev = jnp.broadcast_to(ae[None, None], (q, 1)).astype(jnp.float32)
    out = pl.pallas_call(
        _gp_kernel,
        grid=(q // _GPB,),
        in_specs=[pl.BlockSpec((_GPB, 64), lambda i: (i, 0)),
                  pl.BlockSpec((_GPB, _NN, 64), lambda i: (i, 0, 0)),
                  pl.BlockSpec((_GPB, _NN), lambda i: (i, 0)),
                  pl.BlockSpec((_GPB, 1), lambda i: (i, 0))],
        out_specs=pl.BlockSpec((_GPB, _NN), lambda i: (i, 0)),
        out_shape=jax.ShapeDtypeStruct((q, _NN), jnp.float32),
    )(xs, nxg, nyc, aev)

    y = out[:, 0:1] + ymean
    yvar = ae - out[:, 1]
    return (y, yvar)
